# baseline (speedup 1.0000x reference)
import functools

import jax
import jax.numpy as jnp
from jax import lax
from jax.experimental import pallas as pl
from jax.experimental.pallas import tpu as pltpu
from jax.experimental.pallas import tpu_sc as plsc

N = 10000
N_PAD = 10240
E = 320000
D = 128
DH = D // 2

NC = 2
NS = 16
NW = NC * NS
K = 128
E_PAD = 327680
CH = E_PAD // (NS * K)
EW = E_PAD // NW
RW = EW // K
ZR = 128
RPS = N_PAD // NS
NBUF = 4

_sc_mesh = plsc.VectorSubcoreMesh(core_axis_name="c", subcore_axis_name="s")
_sc_params = pltpu.CompilerParams(needs_layout_passes=False,
                                  use_tc_tiling_on_sc=False)


def _deg_body(src_hbm, dst_hbm, out_hbm, sidx, didx, dego, degi):
    c = lax.axis_index("c")
    s = lax.axis_index("s")
    w = c * NS + s
    pltpu.sync_copy(src_hbm.at[w], sidx)
    pltpu.sync_copy(dst_hbm.at[w], didx)

    @pl.loop(0, N_PAD // 16)
    def _zero(i):
        dego[pl.ds(i * 16, 16)] = jnp.zeros((16,), jnp.float32)
        degi[pl.ds(i * 16, 16)] = jnp.zeros((16,), jnp.float32)

    ones = jnp.ones((16,), jnp.float32)

    @pl.loop(0, RW)
    def _hist(r):
        for l in range(K // 16):
            sv = sidx[r, pl.ds(l * 16, 16)]
            dv = didx[r, pl.ds(l * 16, 16)]
            plsc.addupdate_scatter(dego, [sv], ones)
            plsc.addupdate_scatter(degi, [dv], ones)

    pltpu.sync_copy(dego, out_hbm.at[c, s, 0])
    pltpu.sync_copy(degi, out_hbm.at[c, s, 1])


_deg_kernel = functools.partial(
    pl.kernel,
    out_type=jax.ShapeDtypeStruct((NC, NS, 2, N_PAD), jnp.float32),
    mesh=_sc_mesh,
    scratch_types=[
        pltpu.VMEM((RW, K), jnp.int32),
        pltpu.VMEM((RW, K), jnp.int32),
        pltpu.VMEM((N_PAD,), jnp.float32),
        pltpu.VMEM((N_PAD,), jnp.float32),
    ],
    compiler_params=_sc_params,
)(_deg_body)


def _agg_body(xs_hbm, src_hbm, dst_hbm, out_hbm, sidx, didx, bufs, zbuf, acc,
              gsem, ssem):
    c = lax.axis_index("c")
    s = lax.axis_index("s")
    pltpu.sync_copy(src_hbm.at[s], sidx)
    pltpu.sync_copy(dst_hbm.at[s], didx)

    @pl.loop(0, ZR)
    def _zfill(r):
        for l in range(DH // 16):
            zbuf[r, pl.ds(l * 16, 16)] = jnp.zeros((16,), jnp.float32)

    @pl.loop(0, RPS // ZR)
    def _zacc(t):
        pltpu.sync_copy(zbuf, acc.at[pl.ds(s * RPS + t * ZR, ZR)])

    plsc.subcore_barrier()

    LA = NBUF // 2
    xsc = xs_hbm.at[c]
    for k in range(LA):
        pltpu.async_copy(xsc.at[sidx.at[k]], bufs[k], gsem)

    @pl.loop(0, CH, step=NBUF)
    def _edges(j):
        for k in range(NBUF):
            p = j + k
            buf = bufs[k]
            nbuf = bufs[(k + LA) % NBUF]
            pltpu.make_async_copy(xsc.at[sidx.at[p]], buf, gsem).wait()
            didx_p = didx.at[p]
            pltpu.async_copy(buf, acc.at[didx_p], ssem, add=True)

            @pl.when(p >= LA)
            def _wait_scatter():
                pltpu.make_async_copy(nbuf, acc.at[didx_p], ssem).wait()

            @pl.when(p + LA < CH)
            def _next_gather():
                pltpu.async_copy(xsc.at[sidx.at[p + LA]], nbuf, gsem)

    for k in range(LA):
        pltpu.make_async_copy(bufs[k], acc.at[didx.at[k]], ssem).wait()

    plsc.subcore_barrier()
    pltpu.sync_copy(acc.at[pl.ds(s * RPS, RPS)],
                    out_hbm.at[c, pl.ds(s * RPS, RPS)])


_agg_kernel = functools.partial(
    pl.kernel,
    out_type=jax.ShapeDtypeStruct((NC, N_PAD, DH), jnp.float32),
    mesh=_sc_mesh,
    scratch_types=[
        pltpu.VMEM((CH, K), jnp.int32),
        pltpu.VMEM((CH, K), jnp.int32),
        [pltpu.VMEM((K, DH), jnp.float32) for _ in range(NBUF)],
        pltpu.VMEM((ZR, DH), jnp.float32),
        pltpu.VMEM_SHARED((N_PAD, DH), jnp.float32),
        pltpu.SemaphoreType.DMA,
        pltpu.SemaphoreType.DMA,
    ],
    compiler_params=_sc_params,
)(_agg_body)


def _norm_body(dp_ref, h_ref, xs_ref, nrm_ref):
    d = jnp.sum(dp_ref[...], axis=0)
    nrm = lax.rsqrt(jnp.maximum(d, 1.0))
    nrm_ref[...] = nrm.T
    xs = jnp.transpose(h_ref[...] * nrm[0:1, :], (1, 0))
    xs_ref[0] = xs[:, :DH]
    xs_ref[1] = xs[:, DH:]


def _norm_call(dp, hp, block):
    grid = (N_PAD // block,)
    return pl.pallas_call(
        _norm_body,
        grid=grid,
        in_specs=[
            pl.BlockSpec((NW, 2, block), lambda i: (0, 0, i)),
            pl.BlockSpec((D, block), lambda i: (0, i)),
        ],
        out_specs=[
            pl.BlockSpec((NC, block, DH), lambda i: (0, i, 0)),
            pl.BlockSpec((block, 2), lambda i: (i, 0)),
        ],
        out_shape=[
            jax.ShapeDtypeStruct((NC, N_PAD, DH), jnp.float32),
            jax.ShapeDtypeStruct((N_PAD, 2), jnp.float32),
        ],
    )(dp, hp)


def _mm_body(p_ref, nrm_ref, w_ref, b_ref, o_ref, *, layer1):
    nrm = nrm_ref[...]
    nd = nrm[:, 1:2]
    y = (jnp.dot(p_ref[0] * nd, w_ref[:DH, :],
                 preferred_element_type=jnp.float32)
         + jnp.dot(p_ref[1] * nd, w_ref[DH:, :],
                   preferred_element_type=jnp.float32)
         + b_ref[...])
    if layer1:
        y = jnp.maximum(y, 0.0) * nrm[:, 0:1]
        o_ref[0] = y[:, :DH]
        o_ref[1] = y[:, DH:]
    else:
        o_ref[...] = y


def _mm_call(p, nrm, w, b, layer1, block):
    grid = (N_PAD // block,)
    if layer1:
        out_spec = pl.BlockSpec((NC, block, DH), lambda i: (0, i, 0))
        out_shape = jax.ShapeDtypeStruct((NC, N_PAD, DH), jnp.float32)
    else:
        out_spec = pl.BlockSpec((block, D), lambda i: (i, 0))
        out_shape = jax.ShapeDtypeStruct((N_PAD, D), jnp.float32)
    return pl.pallas_call(
        functools.partial(_mm_body, layer1=layer1),
        grid=grid,
        in_specs=[
            pl.BlockSpec((NC, block, DH), lambda i: (0, i, 0)),
            pl.BlockSpec((block, 2), lambda i: (i, 0)),
            pl.BlockSpec((D, D), lambda i: (0, 0)),
            pl.BlockSpec((1, D), lambda i: (0, 0)),
        ],
        out_specs=out_spec,
        out_shape=out_shape,
    )(p, nrm, w, b)


@jax.jit
def kernel(h, edge_index, W1, b1, W2, b2):
    hp = jnp.pad(h, ((0, 0), (0, N_PAD - N)))
    eip = jnp.pad(edge_index, ((0, 0), (0, E_PAD - E)), constant_values=N)
    srcd = eip[0].reshape(NW, RW, K)
    dstd = eip[1].reshape(NW, RW, K)
    srca = eip[0].reshape(NS, CH, K)
    dsta = eip[1].reshape(NS, CH, K)

    degp = _deg_kernel(srcd, dstd).reshape(NW, 2, N_PAD)
    xs1, nrm = _norm_call(degp, hp, block=1280)

    p1 = _agg_kernel(xs1, srca, dsta)
    xs2 = _mm_call(p1, nrm, W1, b1.reshape(1, D), True, block=1280)

    p2 = _agg_kernel(xs2, srca, dsta)
    out = _mm_call(p2, nrm, W2, b2.reshape(1, D), False, block=1280)

    return jnp.transpose(out[:N], (1, 0))


# speedup vs baseline: 2.2720x; 2.2720x over previous
import functools

import jax
import jax.numpy as jnp
from jax import lax
from jax.experimental import pallas as pl
from jax.experimental.pallas import tpu as pltpu
from jax.experimental.pallas import tpu_sc as plsc

N = 10000
N_PAD = 10240
E = 320000
D = 128
DH = D // 2

NC = 2
NS = 16
NW = NC * NS
K = 128
E_PAD = 327680
CH = E_PAD // (NS * K)
EW = E_PAD // NW
RW = EW // K
ZR = 128
RPS = N_PAD // NS
NBUF = 4

_sc_mesh = plsc.VectorSubcoreMesh(core_axis_name="c", subcore_axis_name="s")
_sc_params = pltpu.CompilerParams(needs_layout_passes=False,
                                  use_tc_tiling_on_sc=False)


def _deg_body(src_hbm, dst_hbm, out_hbm, sidx, didx, dego, degi):
    c = lax.axis_index("c")
    s = lax.axis_index("s")
    w = c * NS + s
    pltpu.sync_copy(src_hbm.at[w], sidx)
    pltpu.sync_copy(dst_hbm.at[w], didx)

    @pl.loop(0, N_PAD // 16)
    def _zero(i):
        dego[pl.ds(i * 16, 16)] = jnp.zeros((16,), jnp.float32)
        degi[pl.ds(i * 16, 16)] = jnp.zeros((16,), jnp.float32)

    ones = jnp.ones((16,), jnp.float32)

    @pl.loop(0, RW)
    def _hist(r):
        for l in range(K // 16):
            sv = sidx[r, pl.ds(l * 16, 16)]
            dv = didx[r, pl.ds(l * 16, 16)]
            plsc.addupdate_scatter(dego, [sv], ones)
            plsc.addupdate_scatter(degi, [dv], ones)

    pltpu.sync_copy(dego, out_hbm.at[c, s, 0])
    pltpu.sync_copy(degi, out_hbm.at[c, s, 1])


_deg_kernel = functools.partial(
    pl.kernel,
    out_type=jax.ShapeDtypeStruct((NC, NS, 2, N_PAD), jnp.float32),
    mesh=_sc_mesh,
    scratch_types=[
        pltpu.VMEM((RW, K), jnp.int32),
        pltpu.VMEM((RW, K), jnp.int32),
        pltpu.VMEM((N_PAD,), jnp.float32),
        pltpu.VMEM((N_PAD,), jnp.float32),
    ],
    compiler_params=_sc_params,
)(_deg_body)


def _agg_body(xs_hbm, src_hbm, dst_hbm, out_hbm, sidx, didx, bufs, zbuf, acc,
              gsem, ssem):
    c = lax.axis_index("c")
    s = lax.axis_index("s")
    pltpu.sync_copy(src_hbm.at[s], sidx)
    pltpu.sync_copy(dst_hbm.at[s], didx)

    @pl.loop(0, ZR)
    def _zfill(r):
        for l in range(DH // 16):
            zbuf[r, pl.ds(l * 16, 16)] = jnp.zeros((16,), jnp.float32)

    @pl.loop(0, RPS // ZR)
    def _zacc(t):
        pltpu.sync_copy(zbuf, acc.at[pl.ds(s * RPS + t * ZR, ZR)])

    plsc.subcore_barrier()

    LA = NBUF // 2
    xsc = xs_hbm.at[c]
    for k in range(LA):
        pltpu.async_copy(xsc.at[sidx.at[k]], bufs[k], gsem)

    @pl.loop(0, CH, step=NBUF)
    def _edges(j):
        for k in range(NBUF):
            p = j + k
            buf = bufs[k]
            nbuf = bufs[(k + LA) % NBUF]
            pltpu.make_async_copy(xsc.at[sidx.at[p]], buf, gsem).wait()
            didx_p = didx.at[p]
            pltpu.async_copy(buf, acc.at[didx_p], ssem, add=True)

            @pl.when(p >= LA)
            def _wait_scatter():
                pltpu.make_async_copy(nbuf, acc.at[didx_p], ssem).wait()

            @pl.when(p + LA < CH)
            def _next_gather():
                pltpu.async_copy(xsc.at[sidx.at[p + LA]], nbuf, gsem)

    for k in range(LA):
        pltpu.make_async_copy(bufs[k], acc.at[didx.at[k]], ssem).wait()

    plsc.subcore_barrier()
    pltpu.sync_copy(acc.at[pl.ds(s * RPS, RPS)],
                    out_hbm.at[c, pl.ds(s * RPS, RPS)])


_agg_kernel = functools.partial(
    pl.kernel,
    out_type=jax.ShapeDtypeStruct((NC, N_PAD, DH), jnp.float32),
    mesh=_sc_mesh,
    scratch_types=[
        pltpu.VMEM((CH, K), jnp.int32),
        pltpu.VMEM((CH, K), jnp.int32),
        [pltpu.VMEM((K, DH), jnp.float32) for _ in range(NBUF)],
        pltpu.VMEM((ZR, DH), jnp.float32),
        pltpu.VMEM_SHARED((N_PAD, DH), jnp.float32),
        pltpu.SemaphoreType.DMA,
        pltpu.SemaphoreType.DMA,
    ],
    compiler_params=_sc_params,
)(_agg_body)


def _norm_body(dp_ref, h_ref, xs_ref, nrm_ref):
    d = jnp.sum(dp_ref[...], axis=0)
    nrm = lax.rsqrt(jnp.maximum(d, 1.0))
    nrm_ref[...] = nrm.T
    xs = jnp.transpose(h_ref[...] * nrm[0:1, :], (1, 0))
    xs_ref[0] = xs[:, :DH]
    xs_ref[1] = xs[:, DH:]


def _norm_call(dp, hp, block):
    grid = (N_PAD // block,)
    return pl.pallas_call(
        _norm_body,
        grid=grid,
        in_specs=[
            pl.BlockSpec((NW, 2, block), lambda i: (0, 0, i)),
            pl.BlockSpec((D, block), lambda i: (0, i)),
        ],
        out_specs=[
            pl.BlockSpec((NC, block, DH), lambda i: (0, i, 0)),
            pl.BlockSpec((block, 2), lambda i: (i, 0)),
        ],
        out_shape=[
            jax.ShapeDtypeStruct((NC, N_PAD, DH), jnp.float32),
            jax.ShapeDtypeStruct((N_PAD, 2), jnp.float32),
        ],
    )(dp, hp)


def _mm_body(p_ref, nrm_ref, w_ref, b_ref, o_ref, *, layer1):
    nrm = nrm_ref[...]
    nd = nrm[:, 1:2]
    y = (jnp.dot(p_ref[0] * nd, w_ref[:DH, :],
                 preferred_element_type=jnp.float32)
         + jnp.dot(p_ref[1] * nd, w_ref[DH:, :],
                   preferred_element_type=jnp.float32)
         + b_ref[...])
    if layer1:
        y = jnp.maximum(y, 0.0) * nrm[:, 0:1]
        o_ref[0] = y[:, :DH]
        o_ref[1] = y[:, DH:]
    else:
        o_ref[...] = y


def _mm_call(p, nrm, w, b, layer1, block):
    grid = (N_PAD // block,)
    if layer1:
        out_spec = pl.BlockSpec((NC, block, DH), lambda i: (0, i, 0))
        out_shape = jax.ShapeDtypeStruct((NC, N_PAD, DH), jnp.float32)
    else:
        out_spec = pl.BlockSpec((block, D), lambda i: (i, 0))
        out_shape = jax.ShapeDtypeStruct((N_PAD, D), jnp.float32)
    return pl.pallas_call(
        functools.partial(_mm_body, layer1=layer1),
        grid=grid,
        in_specs=[
            pl.BlockSpec((NC, block, DH), lambda i: (0, i, 0)),
            pl.BlockSpec((block, 2), lambda i: (i, 0)),
            pl.BlockSpec((D, D), lambda i: (0, 0)),
            pl.BlockSpec((1, D), lambda i: (0, 0)),
        ],
        out_specs=out_spec,
        out_shape=out_shape,
    )(p, nrm, w, b)


@jax.jit
def kernel(h, edge_index, W1, b1, W2, b2):
    hp = jnp.pad(h, ((0, 0), (0, N_PAD - N)))
    dummy = N + (jnp.arange(E_PAD - E, dtype=jnp.int32) % (N_PAD - N))
    eip = jnp.concatenate(
        [edge_index, jnp.broadcast_to(dummy, (2, E_PAD - E))], axis=1)
    srcd = eip[0].reshape(NW, RW, K)
    dstd = eip[1].reshape(NW, RW, K)
    srca = eip[0].reshape(NS, CH, K)
    dsta = eip[1].reshape(NS, CH, K)

    degp = _deg_kernel(srcd, dstd).reshape(NW, 2, N_PAD)
    xs1, nrm = _norm_call(degp, hp, block=1280)

    p1 = _agg_kernel(xs1, srca, dsta)
    xs2 = _mm_call(p1, nrm, W1, b1.reshape(1, D), True, block=1280)

    p2 = _agg_kernel(xs2, srca, dsta)
    out = _mm_call(p2, nrm, W2, b2.reshape(1, D), False, block=1280)

    return jnp.transpose(out[:N], (1, 0))
